# baseline (speedup 1.0000x reference)
import functools

import jax
import jax.numpy as jnp
from jax import lax
from jax.experimental import pallas as pl
from jax.experimental.pallas import tpu as pltpu
from jax.experimental.pallas import tpu_sc as plsc

H = 768
I = 2048
E = 8
T = 2048
K = 2

BLK = 256
CAPB = T // BLK
NREG = E * T
IC = 512
NI = I // IC

NW = 32
TPW = T // NW
CH = 32


def _gate_body(xf_ref, gw_ref, p0_ref, p1_ref, w0_ref, w1_ref, mi_ref,
               mask_ref, rank_ref):
    xf = xf_ref[...]
    logits = lax.dot_general(xf, gw_ref[...], (((1,), (1,)), ((), ())))
    m = jnp.max(logits, axis=1, keepdims=True)
    s = jnp.exp(logits - m)
    p = s / jnp.sum(s, axis=1, keepdims=True)
    lane = lax.broadcasted_iota(jnp.int32, p.shape, 1)
    m1 = jnp.max(p, axis=1, keepdims=True)
    i1 = jnp.min(jnp.where(p == m1, lane, 1000), axis=1, keepdims=True)
    p2 = jnp.where(lane == i1, -1.0, p)
    m2 = jnp.max(p2, axis=1, keepdims=True)
    i2 = jnp.min(jnp.where(p2 == m2, lane, 1000), axis=1, keepdims=True)
    denom = m1 + m2 + 1e-20
    w0_ref[...] = jnp.broadcast_to(m1 / denom, (T, 16))
    w1_ref[...] = jnp.broadcast_to(m2 / denom, (T, 16))

    mask = ((lane == i1) | (lane == i2)).astype(jnp.float32)
    mask_ref[...] = mask
    S = 256
    tri = (lax.broadcasted_iota(jnp.int32, (S, S), 1)
           < lax.broadcasted_iota(jnp.int32, (S, S), 0)).astype(jnp.float32)

    def _strip(k, off):
        sub = mask_ref[pl.ds(k * S, S), :]
        rk = lax.dot_general(tri, sub, (((1,), (0,)), ((), ())))
        rank_ref[pl.ds(k * S, S), :] = rk + jnp.broadcast_to(off, (S, E))
        return off + jnp.sum(sub, axis=0, keepdims=True)

    cnt_f = lax.fori_loop(0, T // S, _strip,
                          jnp.zeros((1, E), jnp.float32))
    rank_x = rank_ref[...]
    rank0 = jnp.sum(rank_x * (lane == i1), axis=1, keepdims=True)
    rank1 = jnp.sum(rank_x * (lane == i2), axis=1, keepdims=True)
    p0_ref[...] = i1 * T + rank0.astype(jnp.int32)
    p1_ref[...] = i2 * T + rank1.astype(jnp.int32)

    cnt = cnt_f.astype(jnp.int32)
    nb = (cnt + BLK - 1) // BLK
    mi_ref[:, 0:E] = nb


def _gate_call(xf, gate_w):
    c2 = lambda i: (0, 0)
    return pl.pallas_call(
        _gate_body,
        grid=(1,),
        in_specs=[pl.BlockSpec((T, H), c2), pl.BlockSpec((E, H), c2)],
        out_specs=[
            pl.BlockSpec((T, 1), c2), pl.BlockSpec((T, 1), c2),
            pl.BlockSpec((T, 16), c2), pl.BlockSpec((T, 16), c2),
            pl.BlockSpec((1, E), c2),
        ],
        out_shape=[
            jax.ShapeDtypeStruct((T, 1), jnp.int32),
            jax.ShapeDtypeStruct((T, 1), jnp.int32),
            jax.ShapeDtypeStruct((T, 16), jnp.float32),
            jax.ShapeDtypeStruct((T, 16), jnp.float32),
            jax.ShapeDtypeStruct((1, E), jnp.int32),
        ],
        scratch_shapes=[pltpu.VMEM((T, E), jnp.float32),
                        pltpu.VMEM((T, E), jnp.float32)],
        compiler_params=pltpu.CompilerParams(
            vmem_limit_bytes=100 * 1024 * 1024),
    )(xf, gate_w)


@functools.lru_cache(maxsize=None)
def _make_dispatch():
    mesh = plsc.VectorSubcoreMesh(core_axis_name="c", subcore_axis_name="s")

    @functools.partial(
        pl.kernel, mesh=mesh,
        out_type=jax.ShapeDtypeStruct((NREG, H), jnp.float32),
        scratch_types=[
            pltpu.VMEM((TPW, H), jnp.float32),
            pltpu.VMEM((TPW,), jnp.int32),
            pltpu.VMEM((TPW,), jnp.int32),
            pltpu.SemaphoreType.DMA,
            pltpu.SemaphoreType.DMA,
        ],
    )
    def _dispatch(xf_hbm, p0_hbm, p1_hbm, disp_hbm, xrows, idx0, idx1,
                  sem0, sem1):
        wid = lax.axis_index("s") * 2 + lax.axis_index("c")
        base = wid * TPW
        pltpu.sync_copy(p0_hbm.at[pl.ds(base, TPW)], idx0)
        pltpu.sync_copy(p1_hbm.at[pl.ds(base, TPW)], idx1)
        pltpu.sync_copy(xf_hbm.at[pl.ds(base, TPW)], xrows)
        c0 = pltpu.async_copy(xrows, disp_hbm.at[idx0], sem0)
        c1 = pltpu.async_copy(xrows, disp_hbm.at[idx1], sem1)
        c0.wait()
        c1.wait()

    return _dispatch


def _ffn3(xin, wg, wu, wd):
    gate = lax.dot_general(xin, wg, (((1,), (1,)), ((), ())))
    up = lax.dot_general(xin, wu, (((1,), (1,)), ((), ())))
    h = jax.nn.silu(gate) * up
    return lax.dot_general(h, wd, (((1,), (1,)), ((), ())))


def _gffn_body(nb_ref, disp_ref, wg_ref, wu_ref, wd_ref, ye_ref):
    e = pl.program_id(0)
    j = pl.program_id(1)

    @pl.when(j < nb_ref[e])
    def _():
        ye_ref[...] = _ffn3(disp_ref[...], wg_ref[0], wu_ref[0], wd_ref[0])


def _gffn_call(nb, disp, Wg, Wu, Wd):
    def _dmap(e, j, nb_ref):
        return (e * CAPB + jnp.minimum(j, jnp.maximum(nb_ref[e] - 1, 0)), 0)

    grid_spec = pltpu.PrefetchScalarGridSpec(
        num_scalar_prefetch=1,
        grid=(E, CAPB),
        in_specs=[
            pl.BlockSpec((BLK, H), _dmap),
            pl.BlockSpec((1, I, H), lambda e, j, nb_ref: (e, 0, 0)),
            pl.BlockSpec((1, I, H), lambda e, j, nb_ref: (e, 0, 0)),
            pl.BlockSpec((1, H, I), lambda e, j, nb_ref: (e, 0, 0)),
        ],
        out_specs=pl.BlockSpec((BLK, H), _dmap),
    )
    return pl.pallas_call(
        _gffn_body,
        grid_spec=grid_spec,
        out_shape=jax.ShapeDtypeStruct((NREG, H), jnp.float32),
        compiler_params=pltpu.CompilerParams(
            dimension_semantics=("arbitrary", "arbitrary"),
            vmem_limit_bytes=100 * 1024 * 1024),
    )(nb, disp, Wg, Wu, Wd)


def _sffn_body(xf_ref, sg_ref, su_ref, sd_ref, out_ref):
    i = pl.program_id(0)
    part = _ffn3(xf_ref[...], sg_ref[...], su_ref[...], sd_ref[...])

    @pl.when(i == 0)
    def _():
        out_ref[...] = part

    @pl.when(i > 0)
    def _():
        out_ref[...] += part


def _sffn_call(xf, Sg, Su, Sd):
    return pl.pallas_call(
        _sffn_body,
        grid=(NI,),
        in_specs=[
            pl.BlockSpec((T, H), lambda i: (0, 0)),
            pl.BlockSpec((IC, H), lambda i: (i, 0)),
            pl.BlockSpec((IC, H), lambda i: (i, 0)),
            pl.BlockSpec((H, IC), lambda i: (0, i)),
        ],
        out_specs=pl.BlockSpec((T, H), lambda i: (0, 0)),
        out_shape=jax.ShapeDtypeStruct((T, H), jnp.float32),
        compiler_params=pltpu.CompilerParams(
            dimension_semantics=("arbitrary",),
            vmem_limit_bytes=100 * 1024 * 1024),
    )(xf, Sg, Su, Sd)


@functools.lru_cache(maxsize=None)
def _make_combine():
    mesh = plsc.VectorSubcoreMesh(core_axis_name="c", subcore_axis_name="s")
    NCH = TPW // CH

    @functools.partial(
        pl.kernel, mesh=mesh,
        out_type=jax.ShapeDtypeStruct((T, H), jnp.float32),
        scratch_types=[
            pltpu.VMEM((NCH, CH, H), jnp.float32),
            pltpu.VMEM((NCH, CH, H), jnp.float32),
            pltpu.VMEM((TPW,), jnp.int32),
            pltpu.VMEM((TPW,), jnp.int32),
            pltpu.VMEM((TPW, 16), jnp.float32),
            pltpu.VMEM((TPW, 16), jnp.float32),
            pltpu.SemaphoreType.DMA,
            pltpu.SemaphoreType.DMA,
            pltpu.SemaphoreType.DMA,
        ],
    )
    def _combine(ye_hbm, p0_hbm, p1_hbm, w0_hbm, w1_hbm, y_hbm,
                 a, b, i0, i1, w0v, w1v, sema, semb, semo):
        wid = lax.axis_index("s") * 2 + lax.axis_index("c")
        base = wid * TPW
        pltpu.sync_copy(p0_hbm.at[pl.ds(base, TPW)], i0)
        pltpu.sync_copy(p1_hbm.at[pl.ds(base, TPW)], i1)
        copies = []
        for c in range(NCH):
            copies.append(pltpu.async_copy(
                ye_hbm.at[i0.at[pl.ds(c * CH, CH)]], a.at[c], sema))
            copies.append(pltpu.async_copy(
                ye_hbm.at[i1.at[pl.ds(c * CH, CH)]], b.at[c], semb))
        pltpu.sync_copy(w0_hbm.at[pl.ds(base, TPW)], w0v)
        pltpu.sync_copy(w1_hbm.at[pl.ds(base, TPW)], w1v)
        outs = []
        for c in range(NCH):
            copies[2 * c].wait()
            copies[2 * c + 1].wait()

            @pl.loop(0, CH)
            def _(i):
                wa = w0v[c * CH + i, :]
                wb = w1v[c * CH + i, :]

                @pl.loop(0, H // 16)
                def _(jj):
                    sl = (c, i, pl.ds(jj * 16, 16))
                    a[sl] = wa * a[sl] + wb * b[sl]

            outs.append(pltpu.async_copy(
                a.at[c], y_hbm.at[pl.ds(base + c * CH, CH)], semo))
        for out in outs:
            out.wait()

    return _combine


def _add_body(a_ref, b_ref, o_ref):
    o_ref[...] = a_ref[...] + b_ref[...]


def _add_call(a, b):
    return pl.pallas_call(
        _add_body,
        grid=(4,),
        in_specs=[pl.BlockSpec((T // 4, H), lambda i: (i, 0)),
                  pl.BlockSpec((T // 4, H), lambda i: (i, 0))],
        out_specs=pl.BlockSpec((T // 4, H), lambda i: (i, 0)),
        out_shape=jax.ShapeDtypeStruct((T, H), jnp.float32),
        compiler_params=pltpu.CompilerParams(
            dimension_semantics=("arbitrary",)),
    )(a, b)


@jax.jit
def kernel(x, gate_w, Wg, Wu, Wd, Sg, Su, Sd):
    bsz, seq_len, h = x.shape
    xf = x.reshape(T, H)
    p0, p1, w0r, w1r, mi = _gate_call(xf, gate_w)
    p0f = p0.reshape(T)
    p1f = p1.reshape(T)
    nb = mi[0]
    disp = _make_dispatch()(xf, p0f, p1f)
    ys = _sffn_call(xf, Sg, Su, Sd)
    ye = _gffn_call(nb, disp, Wg, Wu, Wd)
    yr = _make_combine()(ye, p0f, p1f, w0r, w1r)
    y = _add_call(yr, ys)
    return y.reshape(bsz, seq_len, h)


# speedup vs baseline: 1.2117x; 1.2117x over previous
import functools

import jax
import jax.numpy as jnp
from jax import lax
from jax.experimental import pallas as pl
from jax.experimental.pallas import tpu as pltpu
from jax.experimental.pallas import tpu_sc as plsc

H = 768
I = 2048
E = 8
T = 2048
K = 2

BLK = 512
CAPB = T // BLK
NREG = E * T
GR = (K * T) // BLK + E
IC = 512
NI = I // IC

NW = 32
TPW = T // NW
CH = 32


def _gate_body(xf_ref, gw_ref, p0_ref, p1_ref, w0_ref, w1_ref, mi_ref,
               mask_ref, rank_ref):
    xf = xf_ref[...]
    logits = lax.dot_general(xf, gw_ref[...], (((1,), (1,)), ((), ())))
    m = jnp.max(logits, axis=1, keepdims=True)
    s = jnp.exp(logits - m)
    p = s / jnp.sum(s, axis=1, keepdims=True)
    lane = lax.broadcasted_iota(jnp.int32, p.shape, 1)
    m1 = jnp.max(p, axis=1, keepdims=True)
    i1 = jnp.min(jnp.where(p == m1, lane, 1000), axis=1, keepdims=True)
    p2 = jnp.where(lane == i1, -1.0, p)
    m2 = jnp.max(p2, axis=1, keepdims=True)
    i2 = jnp.min(jnp.where(p2 == m2, lane, 1000), axis=1, keepdims=True)
    denom = m1 + m2 + 1e-20
    w0_ref[...] = jnp.broadcast_to(m1 / denom, (T, 16))
    w1_ref[...] = jnp.broadcast_to(m2 / denom, (T, 16))

    mask = ((lane == i1) | (lane == i2)).astype(jnp.float32)
    mask_ref[...] = mask
    S = 256
    tri = (lax.broadcasted_iota(jnp.int32, (S, S), 1)
           < lax.broadcasted_iota(jnp.int32, (S, S), 0)).astype(jnp.float32)

    def _strip(k, off):
        sub = mask_ref[pl.ds(k * S, S), :]
        rk = lax.dot_general(tri, sub, (((1,), (0,)), ((), ())))
        rank_ref[pl.ds(k * S, S), :] = rk + jnp.broadcast_to(off, (S, E))
        return off + jnp.sum(sub, axis=0, keepdims=True)

    cnt_f = lax.fori_loop(0, T // S, _strip,
                          jnp.zeros((1, E), jnp.float32))
    rank_x = rank_ref[...]
    rank0 = jnp.sum(rank_x * (lane == i1), axis=1, keepdims=True)
    rank1 = jnp.sum(rank_x * (lane == i2), axis=1, keepdims=True)
    p0_ref[...] = i1 * T + rank0.astype(jnp.int32)
    p1_ref[...] = i2 * T + rank1.astype(jnp.int32)

    cnt = cnt_f.astype(jnp.int32)
    nb = (cnt + BLK - 1) // BLK
    e_row = lax.broadcasted_iota(jnp.int32, (E, E), 0)
    e_col = lax.broadcasted_iota(jnp.int32, (E, E), 1)
    le = (e_row <= e_col).astype(jnp.float32)
    cumnb = lax.dot_general(nb.astype(jnp.float32), le,
                            (((1,), (0,)), ((), ()))).astype(jnp.int32)
    total = cumnb[0:1, E - 1:E]
    gl = lax.broadcasted_iota(jnp.int32, (1, 64), 1)
    g_eff = jnp.minimum(gl, total - 1)
    ge_b = jnp.broadcast_to(g_eff, (E, 64))
    cum_col = jnp.broadcast_to(
        jnp.transpose(cumnb.astype(jnp.float32)).astype(jnp.int32), (E, 64))
    eg = jnp.sum((ge_b >= cum_col).astype(jnp.int32), axis=0, keepdims=True)
    cumx_col = cum_col - jnp.broadcast_to(
        jnp.transpose(nb.astype(jnp.float32)).astype(jnp.int32), (E, 64))
    e_iota = lax.broadcasted_iota(jnp.int32, (E, 64), 0)
    eg_b = jnp.broadcast_to(eg, (E, 64))
    cumx_sel = jnp.sum(jnp.where(e_iota == eg_b, cumx_col, 0), axis=0,
                       keepdims=True)
    brow = eg * CAPB + (g_eff - cumx_sel)
    mi_ref[:, 0:64] = eg
    mi_ref[:, 64:128] = brow
    mi_ref[:, 128:192] = jnp.broadcast_to(total, (1, 64))


def _gate_call(xf, gate_w):
    c2 = lambda i: (0, 0)
    return pl.pallas_call(
        _gate_body,
        grid=(1,),
        in_specs=[pl.BlockSpec((T, H), c2), pl.BlockSpec((E, H), c2)],
        out_specs=[
            pl.BlockSpec((T, 1), c2), pl.BlockSpec((T, 1), c2),
            pl.BlockSpec((T, 16), c2), pl.BlockSpec((T, 16), c2),
            pl.BlockSpec((1, 192), c2),
        ],
        out_shape=[
            jax.ShapeDtypeStruct((T, 1), jnp.int32),
            jax.ShapeDtypeStruct((T, 1), jnp.int32),
            jax.ShapeDtypeStruct((T, 16), jnp.float32),
            jax.ShapeDtypeStruct((T, 16), jnp.float32),
            jax.ShapeDtypeStruct((1, 192), jnp.int32),
        ],
        scratch_shapes=[pltpu.VMEM((T, E), jnp.float32),
                        pltpu.VMEM((T, E), jnp.float32)],
        compiler_params=pltpu.CompilerParams(
            vmem_limit_bytes=100 * 1024 * 1024),
    )(xf, gate_w)


@functools.lru_cache(maxsize=None)
def _make_dispatch():
    mesh = plsc.VectorSubcoreMesh(core_axis_name="c", subcore_axis_name="s")

    @functools.partial(
        pl.kernel, mesh=mesh,
        out_type=jax.ShapeDtypeStruct((NREG, H), jnp.float32),
        scratch_types=[
            pltpu.VMEM((TPW, H), jnp.float32),
            pltpu.VMEM((TPW,), jnp.int32),
            pltpu.VMEM((TPW,), jnp.int32),
            pltpu.SemaphoreType.DMA,
            pltpu.SemaphoreType.DMA,
        ],
    )
    def _dispatch(xf_hbm, p0_hbm, p1_hbm, disp_hbm, xrows, idx0, idx1,
                  sem0, sem1):
        wid = lax.axis_index("s") * 2 + lax.axis_index("c")
        base = wid * TPW
        pltpu.sync_copy(p0_hbm.at[pl.ds(base, TPW)], idx0)
        pltpu.sync_copy(p1_hbm.at[pl.ds(base, TPW)], idx1)
        pltpu.sync_copy(xf_hbm.at[pl.ds(base, TPW)], xrows)
        c0 = pltpu.async_copy(xrows, disp_hbm.at[idx0], sem0)
        c1 = pltpu.async_copy(xrows, disp_hbm.at[idx1], sem1)
        c0.wait()
        c1.wait()

    return _dispatch


def _ffn3(xin, wg, wu, wd):
    gate = lax.dot_general(xin, wg, (((1,), (1,)), ((), ())))
    up = lax.dot_general(xin, wu, (((1,), (1,)), ((), ())))
    h = jax.nn.silu(gate) * up
    return lax.dot_general(h, wd, (((1,), (1,)), ((), ())))


def _gffn_body(be_ref, br_ref, tot_ref, disp_ref, wg_ref, wu_ref, wd_ref,
               ye_ref):
    g = pl.program_id(0)

    @pl.when(g < tot_ref[0])
    def _():
        ye_ref[...] = _ffn3(disp_ref[...], wg_ref[0], wu_ref[0], wd_ref[0])


def _gffn_call(be, br, tot, disp, Wg, Wu, Wd):
    grid_spec = pltpu.PrefetchScalarGridSpec(
        num_scalar_prefetch=3,
        grid=(GR,),
        in_specs=[
            pl.BlockSpec((BLK, H), lambda g, be, br, tot: (br[g], 0)),
            pl.BlockSpec((1, I, H), lambda g, be, br, tot: (be[g], 0, 0)),
            pl.BlockSpec((1, I, H), lambda g, be, br, tot: (be[g], 0, 0)),
            pl.BlockSpec((1, H, I), lambda g, be, br, tot: (be[g], 0, 0)),
        ],
        out_specs=pl.BlockSpec((BLK, H), lambda g, be, br, tot: (br[g], 0)),
    )
    return pl.pallas_call(
        _gffn_body,
        grid_spec=grid_spec,
        out_shape=jax.ShapeDtypeStruct((NREG, H), jnp.float32),
        compiler_params=pltpu.CompilerParams(
            dimension_semantics=("arbitrary",),
            vmem_limit_bytes=100 * 1024 * 1024),
    )(be, br, tot, disp, Wg, Wu, Wd)


def _sffn_body(xf_ref, sg_ref, su_ref, sd_ref, out_ref):
    i = pl.program_id(0)
    part = _ffn3(xf_ref[...], sg_ref[...], su_ref[...], sd_ref[...])

    @pl.when(i == 0)
    def _():
        out_ref[...] = part

    @pl.when(i > 0)
    def _():
        out_ref[...] += part


def _sffn_call(xf, Sg, Su, Sd):
    return pl.pallas_call(
        _sffn_body,
        grid=(NI,),
        in_specs=[
            pl.BlockSpec((T, H), lambda i: (0, 0)),
            pl.BlockSpec((IC, H), lambda i: (i, 0)),
            pl.BlockSpec((IC, H), lambda i: (i, 0)),
            pl.BlockSpec((H, IC), lambda i: (0, i)),
        ],
        out_specs=pl.BlockSpec((T, H), lambda i: (0, 0)),
        out_shape=jax.ShapeDtypeStruct((T, H), jnp.float32),
        compiler_params=pltpu.CompilerParams(
            dimension_semantics=("arbitrary",),
            vmem_limit_bytes=100 * 1024 * 1024),
    )(xf, Sg, Su, Sd)


@functools.lru_cache(maxsize=None)
def _make_combine():
    mesh = plsc.VectorSubcoreMesh(core_axis_name="c", subcore_axis_name="s")
    NCH = TPW // CH

    @functools.partial(
        pl.kernel, mesh=mesh,
        out_type=jax.ShapeDtypeStruct((T, H), jnp.float32),
        scratch_types=[
            pltpu.VMEM((NCH, CH, H), jnp.float32),
            pltpu.VMEM((NCH, CH, H), jnp.float32),
            pltpu.VMEM((TPW,), jnp.int32),
            pltpu.VMEM((TPW,), jnp.int32),
            pltpu.VMEM((TPW, 16), jnp.float32),
            pltpu.VMEM((TPW, 16), jnp.float32),
            pltpu.SemaphoreType.DMA,
            pltpu.SemaphoreType.DMA,
            pltpu.SemaphoreType.DMA,
        ],
    )
    def _combine(ye_hbm, p0_hbm, p1_hbm, w0_hbm, w1_hbm, y_hbm,
                 a, b, i0, i1, w0v, w1v, sema, semb, semo):
        wid = lax.axis_index("s") * 2 + lax.axis_index("c")
        base = wid * TPW
        pltpu.sync_copy(p0_hbm.at[pl.ds(base, TPW)], i0)
        pltpu.sync_copy(p1_hbm.at[pl.ds(base, TPW)], i1)
        copies = []
        for c in range(NCH):
            copies.append(pltpu.async_copy(
                ye_hbm.at[i0.at[pl.ds(c * CH, CH)]], a.at[c], sema))
            copies.append(pltpu.async_copy(
                ye_hbm.at[i1.at[pl.ds(c * CH, CH)]], b.at[c], semb))
        pltpu.sync_copy(w0_hbm.at[pl.ds(base, TPW)], w0v)
        pltpu.sync_copy(w1_hbm.at[pl.ds(base, TPW)], w1v)
        outs = []
        for c in range(NCH):
            copies[2 * c].wait()
            copies[2 * c + 1].wait()

            @pl.loop(0, CH)
            def _(i):
                wa = w0v[c * CH + i, :]
                wb = w1v[c * CH + i, :]

                @pl.loop(0, H // 16)
                def _(jj):
                    sl = (c, i, pl.ds(jj * 16, 16))
                    a[sl] = wa * a[sl] + wb * b[sl]

            outs.append(pltpu.async_copy(
                a.at[c], y_hbm.at[pl.ds(base + c * CH, CH)], semo))
        for out in outs:
            out.wait()

    return _combine


def _add_body(a_ref, b_ref, o_ref):
    o_ref[...] = a_ref[...] + b_ref[...]


def _add_call(a, b):
    return pl.pallas_call(
        _add_body,
        grid=(4,),
        in_specs=[pl.BlockSpec((T // 4, H), lambda i: (i, 0)),
                  pl.BlockSpec((T // 4, H), lambda i: (i, 0))],
        out_specs=pl.BlockSpec((T // 4, H), lambda i: (i, 0)),
        out_shape=jax.ShapeDtypeStruct((T, H), jnp.float32),
        compiler_params=pltpu.CompilerParams(
            dimension_semantics=("arbitrary",)),
    )(a, b)


@jax.jit
def kernel(x, gate_w, Wg, Wu, Wd, Sg, Su, Sd):
    bsz, seq_len, h = x.shape
    xf = x.reshape(T, H)
    p0, p1, w0r, w1r, mi = _gate_call(xf, gate_w)
    p0f = p0.reshape(T)
    p1f = p1.reshape(T)
    be = mi[0, 0:GR]
    br = mi[0, 64:64 + GR]
    tot = mi[0, 128:129]
    disp = _make_dispatch()(xf, p0f, p1f)
    ys = _sffn_call(xf, Sg, Su, Sd)
    ye = _gffn_call(be, br, tot, disp, Wg, Wu, Wd)
    yr = _make_combine()(ye, p0f, p1f, w0r, w1r)
    y = _add_call(yr, ys)
    return y.reshape(bsz, seq_len, h)
